# Initial kernel scaffold; baseline (speedup 1.0000x reference)
import jax
import jax.numpy as jnp
from jax.experimental import pallas as pl


def kernel(ind_1, elems, ind_2, diff, params):
    raise NotImplementedError("write your pallas kernel here")


# speedup vs baseline: 1.0098x; 1.0098x over previous
import functools
import jax
import jax.numpy as jnp
from jax import lax
from jax.experimental import pallas as pl
from jax.experimental.pallas import tpu as pltpu

RC = 5.0
N_BASIS = 10
DEPTH = 4
D = 16
N_TYPES = 4

E_BLK = 2000


def _mm(a, b):
    return jax.lax.dot_general(
        a, b, (((1,), (0,)), ((), ())),
        precision=jax.lax.Precision.DEFAULT,
        preferred_element_type=jnp.float32)


def _mm_exact(a, b):
    return jax.lax.dot_general(
        a, b, (((1,), (0,)), ((), ())),
        precision=jax.lax.Precision.HIGHEST,
        preferred_element_type=jnp.float32)


def _fc_body(diff_ref, fc_ref):
    d = diff_ref[...]
    dist = jnp.sqrt(jnp.sum(d * d, axis=1, keepdims=True))
    x = jnp.clip(dist / RC, 0.0, 1.0)
    fc = 0.5 * (jnp.cos(jnp.pi * x) + 1.0)
    fc_ref[...] = jnp.where(dist > 0.0, fc, 0.0)


def _compute_fc(diff):
    E = diff.shape[0]
    grid = E // E_BLK
    return pl.pallas_call(
        _fc_body,
        grid=(grid,),
        in_specs=[pl.BlockSpec((E_BLK, 3), lambda i: (i, 0))],
        out_specs=pl.BlockSpec((E_BLK, 1), lambda i: (i, 0)),
        out_shape=jax.ShapeDtypeStruct((E, 1), jnp.float32),
    )(diff)


def _edge_body(g_ref, fc_ref, w2_ref, b2_ref, wii0_ref, wii1_ref, inter_ref):
    x1 = jnp.tanh(g_ref[...])
    h = jnp.tanh(_mm(x1, w2_ref[...]) + b2_ref[...])
    blk = x1.shape[0]
    k = lax.broadcasted_iota(jnp.int32, (blk, D * N_BASIS), 1)
    ex = ((k % N_BASIS) + 1).astype(jnp.float32)
    fc = fc_ref[...]
    bt = jnp.exp(ex * jnp.log(fc))
    hw = h * bt
    row = lax.broadcasted_iota(jnp.int32, (D * N_BASIS, D), 0) // N_BASIS
    col = lax.broadcasted_iota(jnp.int32, (D * N_BASIS, D), 1)
    gsel = (row == col).astype(jnp.float32)
    inter = _mm_exact(hw, gsel)
    y = jnp.tanh(_mm(inter, wii0_ref[...]))
    inter_ref[...] = jnp.tanh(_mm(y, wii1_ref[...]))


def _edge_stage(g, fc, w2, b2, wii0, wii1):
    E = g.shape[0]
    grid = E // E_BLK
    wspec = lambda shape: pl.BlockSpec(shape, lambda i: (0, 0))
    return pl.pallas_call(
        _edge_body,
        grid=(grid,),
        in_specs=[
            pl.BlockSpec((E_BLK, D), lambda i: (i, 0)),
            pl.BlockSpec((E_BLK, 1), lambda i: (i, 0)),
            wspec((D, D * N_BASIS)),
            wspec((1, D * N_BASIS)),
            wspec((D, D)),
            wspec((D, D)),
        ],
        out_specs=pl.BlockSpec((E_BLK, D), lambda i: (i, 0)),
        out_shape=jax.ShapeDtypeStruct((E, D), jnp.float32),
    )(g, fc, w2, b2, wii0, wii1)


def _atom0_body(elems_ref, wp0_ref, bp0_ref, wp1_ref, bp1_ref,
                w1a_ref, w1b_ref, b1_ref, a_ref, b_ref):
    el = elems_ref[...]
    n = el.shape[0]
    types = lax.broadcasted_iota(jnp.int32, (n, N_TYPES), 1)
    onehot = (el == types).astype(jnp.float32)
    p = jnp.tanh(_mm(onehot, wp0_ref[...]) + bp0_ref[...])
    p = jnp.tanh(_mm(p, wp1_ref[...]) + bp1_ref[...])
    a_ref[...] = _mm(p, w1a_ref[...]) + b1_ref[...]
    b_ref[...] = _mm(p, w1b_ref[...])


N_BLK = 2000


def _atom0_stage(elems2d, wp0, bp0, wp1, bp1, w1a, w1b, b1):
    N = elems2d.shape[0]
    blk = lambda s: pl.BlockSpec((N_BLK, s), lambda i: (i, 0))
    wsp = lambda s: pl.BlockSpec(s, lambda i: (0, 0))
    return pl.pallas_call(
        _atom0_body,
        grid=(N // N_BLK,),
        in_specs=[blk(1), wsp((N_TYPES, D)), wsp((1, D)),
                  wsp((D, D)), wsp((1, D)),
                  wsp((D, D)), wsp((D, D)), wsp((1, D))],
        out_specs=(blk(D), blk(D)),
        out_shape=(jax.ShapeDtypeStruct((N, D), jnp.float32),
                   jax.ShapeDtypeStruct((N, D), jnp.float32)),
    )(elems2d, wp0, bp0, wp1, bp1, w1a, w1b, b1)


def _atom_body(first, last,
               np_ref, prev_ref, acc_ref,
               wo0_ref, bo0_ref, wo1_ref, bo1_ref, wof_ref, bof_ref,
               wp0_ref, bp0_ref, wp1_ref, bp1_ref,
               w1a_ref, w1b_ref, b1_ref,
               prop_ref, out_ref, a_ref, b_ref):
    new_prop = np_ref[...]
    if first:
        prop = new_prop
        acc = jnp.zeros_like(out_ref)
    else:
        prop = prev_ref[...] + new_prop
        acc = acc_ref[...]
    o = jnp.tanh(_mm(prop, wo0_ref[...]) + bo0_ref[...])
    o = jnp.tanh(_mm(o, wo1_ref[...]) + bo1_ref[...])
    o = _mm(o, wof_ref[...]) + bof_ref[...]
    prop_ref[...] = prop
    out_ref[...] = acc + o
    if not last:
        p = jnp.tanh(_mm(prop, wp0_ref[...]) + bp0_ref[...])
        p = jnp.tanh(_mm(p, wp1_ref[...]) + bp1_ref[...])
        a_ref[...] = _mm(p, w1a_ref[...]) + b1_ref[...]
        b_ref[...] = _mm(p, w1b_ref[...])
    else:
        a_ref[...] = jnp.zeros_like(a_ref)
        b_ref[...] = jnp.zeros_like(b_ref)


def _atom_stage(first, last, new_prop, prev_prop, out_acc,
                wo0, bo0, wo1, bo1, wof, bof,
                wp0, bp0, wp1, bp1, w1a, w1b, b1):
    N = new_prop.shape[0]
    blk = lambda s: pl.BlockSpec((N_BLK, s), lambda i: (i, 0))
    full = lambda s: pl.BlockSpec(s, lambda i: (0, 0))
    body = functools.partial(_atom_body, first, last)
    return pl.pallas_call(
        body,
        grid=(N // N_BLK,),
        in_specs=[blk(D), blk(D), blk(1),
                  full((D, D)), full((1, D)), full((D, D)), full((1, D)),
                  full((D, 1)), full((1, 1)),
                  full((D, D)), full((1, D)), full((D, D)), full((1, D)),
                  full((D, D)), full((D, D)), full((1, D))],
        out_specs=(blk(D), blk(1), blk(D), blk(D)),
        out_shape=(jax.ShapeDtypeStruct((N, D), jnp.float32),
                   jax.ShapeDtypeStruct((N, 1), jnp.float32),
                   jax.ShapeDtypeStruct((N, D), jnp.float32),
                   jax.ShapeDtypeStruct((N, D), jnp.float32)),
    )(new_prop, prev_prop, out_acc,
      wo0, bo0, wo1, bo1, wof, bof,
      wp0, bp0, wp1, bp1, w1a, w1b, b1)


def _gather_stage(A, B, i_idx, j_idx):
    return jnp.take(A, i_idx, axis=0) + jnp.take(B, j_idx, axis=0)


def _scatter_stage(inter, i_idx, n_atoms):
    return jax.ops.segment_sum(inter, i_idx, num_segments=n_atoms)


def _row(b):
    return b.reshape(1, -1)


def kernel(ind_1, elems, ind_2, diff, params):
    n_atoms = elems.shape[0]
    i_idx = jnp.asarray(ind_2[:, 0], dtype=jnp.int32)
    j_idx = jnp.asarray(ind_2[:, 1], dtype=jnp.int32)
    elems2d = elems.astype(jnp.int32).reshape(n_atoms, 1)

    fc = _compute_fc(diff)

    W = []
    for d in range(DEPTH):
        (wp0, bp0), (wp1, bp1) = params["pp"][d]
        (w1, b1), (w2, b2) = params["pi"][d]
        (wii0, _), (wii1, _) = params["ii"][d]
        (wo0, bo0), (wo1, bo1), (wof, bof) = params["out"][d]
        W.append(dict(
            wp0=wp0, bp0=_row(bp0), wp1=wp1, bp1=_row(bp1),
            w1a=w1[:D], w1b=w1[D:], b1=_row(b1),
            w2=w2, b2=_row(b2),
            wii0=wii0, wii1=wii1,
            wo0=wo0, bo0=_row(bo0), wo1=wo1, bo1=_row(bo1),
            wof=wof, bof=_row(bof),
        ))

    A, B = _atom0_stage(elems2d, W[0]["wp0"], W[0]["bp0"],
                        W[0]["wp1"], W[0]["bp1"],
                        W[0]["w1a"], W[0]["w1b"], W[0]["b1"])

    prop = jnp.zeros((n_atoms, D), jnp.float32)
    out_acc = jnp.zeros((n_atoms, 1), jnp.float32)
    for d in range(DEPTH):
        g = _gather_stage(A, B, i_idx, j_idx)
        inter = _edge_stage(g, fc, W[d]["w2"], W[d]["b2"],
                            W[d]["wii0"], W[d]["wii1"])
        new_prop = _scatter_stage(inter, i_idx, n_atoms)
        nxt = W[d + 1] if d + 1 < DEPTH else W[d]
        prop, out_acc, A, B = _atom_stage(
            d == 0, d == DEPTH - 1, new_prop, prop, out_acc,
            W[d]["wo0"], W[d]["bo0"], W[d]["wo1"], W[d]["bo1"],
            W[d]["wof"], W[d]["bof"],
            nxt["wp0"], nxt["bp0"], nxt["wp1"], nxt["bp1"],
            nxt["w1a"], nxt["w1b"], nxt["b1"])

    return out_acc[:, 0]


# speedup vs baseline: 2.3789x; 2.3558x over previous
import functools
import jax
import jax.numpy as jnp
from jax import lax
from jax.experimental import pallas as pl
from jax.experimental.pallas import tpu as pltpu
from jax.experimental.pallas import tpu_sc as plsc

RC = 5.0
N_BASIS = 10
DEPTH = 4
D = 16
N_TYPES = 4

E_BLK = 2000


def _mm(a, b):
    return jax.lax.dot_general(
        a, b, (((1,), (0,)), ((), ())),
        precision=jax.lax.Precision.DEFAULT,
        preferred_element_type=jnp.float32)


def _mm_exact(a, b):
    return jax.lax.dot_general(
        a, b, (((1,), (0,)), ((), ())),
        precision=jax.lax.Precision.HIGHEST,
        preferred_element_type=jnp.float32)


def _fc_body(diff_ref, fc_ref):
    d = diff_ref[...]
    dist = jnp.sqrt(jnp.sum(d * d, axis=1, keepdims=True))
    x = jnp.clip(dist / RC, 0.0, 1.0)
    fc = 0.5 * (jnp.cos(jnp.pi * x) + 1.0)
    fc_ref[...] = jnp.where(dist > 0.0, fc, 0.0)


def _compute_fc(diff):
    E = diff.shape[0]
    grid = E // E_BLK
    return pl.pallas_call(
        _fc_body,
        grid=(grid,),
        in_specs=[pl.BlockSpec((E_BLK, 3), lambda i: (i, 0))],
        out_specs=pl.BlockSpec((E_BLK, 1), lambda i: (i, 0)),
        out_shape=jax.ShapeDtypeStruct((E, 1), jnp.float32),
    )(diff)


def _edge_body(g_ref, fc_ref, w2_ref, b2_ref, wii0_ref, wii1_ref,
               inter_ref):
    x1 = jnp.tanh(g_ref[...])
    h = jnp.tanh(_mm(x1, w2_ref[...]) + b2_ref[...])
    blk = x1.shape[0]
    k = lax.broadcasted_iota(jnp.int32, (blk, D * N_BASIS), 1)
    ex = ((k % N_BASIS) + 1).astype(jnp.float32)
    fc = fc_ref[...]
    bt = jnp.exp(ex * jnp.log(fc))
    hw = h * bt
    row = lax.broadcasted_iota(jnp.int32, (D * N_BASIS, D), 0) // N_BASIS
    col = lax.broadcasted_iota(jnp.int32, (D * N_BASIS, D), 1)
    gsel = (row == col).astype(jnp.float32)
    inter = _mm_exact(hw, gsel)
    y = jnp.tanh(_mm(inter, wii0_ref[...]))
    inter_ref[...] = jnp.tanh(_mm(y, wii1_ref[...]))


def _edge_stage(g, fc, w2, b2, wii0, wii1):
    E = g.shape[0]
    grid = E // E_BLK
    wspec = lambda shape: pl.BlockSpec(shape, lambda i: (0, 0))
    return pl.pallas_call(
        _edge_body,
        grid=(grid,),
        in_specs=[
            pl.BlockSpec((E_BLK, D), lambda i: (i, 0)),
            pl.BlockSpec((E_BLK, 1), lambda i: (i, 0)),
            wspec((D, D * N_BASIS)),
            wspec((1, D * N_BASIS)),
            wspec((D, D)),
            wspec((D, D)),
        ],
        out_specs=pl.BlockSpec((E_BLK, D), lambda i: (i, 0)),
        out_shape=jax.ShapeDtypeStruct((E, D), jnp.float32),
    )(g, fc, w2, b2, wii0, wii1)


def _atom0_body(elems_ref, wp0_ref, bp0_ref, wp1_ref, bp1_ref,
                w1a_ref, w1b_ref, b1_ref, t_ref):
    el = elems_ref[...]
    n = el.shape[0]
    types = lax.broadcasted_iota(jnp.int32, (n, N_TYPES), 1)
    onehot = (el == types).astype(jnp.float32)
    p = jnp.tanh(_mm(onehot, wp0_ref[...]) + bp0_ref[...])
    p = jnp.tanh(_mm(p, wp1_ref[...]) + bp1_ref[...])
    a = _mm(p, w1a_ref[...]) + b1_ref[...]
    b = _mm(p, w1b_ref[...])
    t_ref[...] = jnp.concatenate(
        [a, b, jnp.zeros((n, 128 - 2 * D), jnp.float32)], axis=1)


N_BLK = 2000


def _atom0_stage(elems2d, wp0, bp0, wp1, bp1, w1a, w1b, b1):
    N = elems2d.shape[0]
    blk = lambda s: pl.BlockSpec((N_BLK, s), lambda i: (i, 0))
    wsp = lambda s: pl.BlockSpec(s, lambda i: (0, 0))
    return pl.pallas_call(
        _atom0_body,
        grid=(N // N_BLK,),
        in_specs=[blk(1), wsp((N_TYPES, D)), wsp((1, D)),
                  wsp((D, D)), wsp((1, D)),
                  wsp((D, D)), wsp((D, D)), wsp((1, D))],
        out_specs=blk(128),
        out_shape=jax.ShapeDtypeStruct((N, 128), jnp.float32),
    )(elems2d, wp0, bp0, wp1, bp1, w1a, w1b, b1)


def _atom_body(first, last,
               np0_ref, np1_ref, prev_ref, acc_ref,
               wo0_ref, bo0_ref, wo1_ref, bo1_ref, wof_ref, bof_ref,
               wp0_ref, bp0_ref, wp1_ref, bp1_ref,
               w1a_ref, w1b_ref, b1_ref,
               prop_ref, out_ref, t_ref):
    new_prop = np0_ref[...][:, :D] + np1_ref[...][:, :D]
    if first:
        prop = new_prop
        acc = jnp.zeros_like(out_ref)
    else:
        prop = prev_ref[...] + new_prop
        acc = acc_ref[...]
    o = jnp.tanh(_mm(prop, wo0_ref[...]) + bo0_ref[...])
    o = jnp.tanh(_mm(o, wo1_ref[...]) + bo1_ref[...])
    o = _mm(o, wof_ref[...]) + bof_ref[...]
    prop_ref[...] = prop
    out_ref[...] = acc + o
    n = prop.shape[0]
    if not last:
        p = jnp.tanh(_mm(prop, wp0_ref[...]) + bp0_ref[...])
        p = jnp.tanh(_mm(p, wp1_ref[...]) + bp1_ref[...])
        a = _mm(p, w1a_ref[...]) + b1_ref[...]
        b = _mm(p, w1b_ref[...])
        t_ref[...] = jnp.concatenate(
            [a, b, jnp.zeros((n, 128 - 2 * D), jnp.float32)], axis=1)
    else:
        t_ref[...] = jnp.zeros_like(t_ref)


def _atom_stage(first, last, np0, np1, prev_prop, out_acc,
                wo0, bo0, wo1, bo1, wof, bof,
                wp0, bp0, wp1, bp1, w1a, w1b, b1):
    N = np0.shape[0]
    blk = lambda s: pl.BlockSpec((N_BLK, s), lambda i: (i, 0))
    full = lambda s: pl.BlockSpec(s, lambda i: (0, 0))
    body = functools.partial(_atom_body, first, last)
    return pl.pallas_call(
        body,
        grid=(N // N_BLK,),
        in_specs=[blk(128), blk(128), blk(D), blk(1),
                  full((D, D)), full((1, D)), full((D, D)), full((1, D)),
                  full((D, 1)), full((1, 1)),
                  full((D, D)), full((1, D)), full((D, D)), full((1, D)),
                  full((D, D)), full((D, D)), full((1, D))],
        out_specs=(blk(D), blk(1), blk(128)),
        out_shape=(jax.ShapeDtypeStruct((N, D), jnp.float32),
                   jax.ShapeDtypeStruct((N, 1), jnp.float32),
                   jax.ShapeDtypeStruct((N, 128), jnp.float32)),
    )(np0, np1, prev_prop, out_acc,
      wo0, bo0, wo1, bo1, wof, bof,
      wp0, bp0, wp1, bp1, w1a, w1b, b1)


_SC_MESH = plsc.VectorSubcoreMesh(core_axis_name="c", subcore_axis_name="s")
N_SC = 2
N_TILES = 16
N_WORKERS = N_SC * N_TILES
GCH = 200
SCH = 80
NPAD = 10240


def _gather_body(t_hbm, i_hbm, j_hbm, g_hbm,
                 idx_v, jdx_v, ra_v, rb_v, g_v, semi, semj):
    wid = lax.axis_index("s") * N_SC + lax.axis_index("c")
    epw = i_hbm.shape[0] // N_WORKERS

    def chunk(c, carry):
        base = wid * epw + c * GCH
        pltpu.sync_copy(i_hbm.at[pl.ds(base, GCH)], idx_v)
        pltpu.sync_copy(j_hbm.at[pl.ds(base, GCH)], jdx_v)
        cpa = pltpu.async_copy(t_hbm.at[idx_v], ra_v, semi)
        cpb = pltpu.async_copy(t_hbm.at[jdx_v], rb_v, semj)
        cpa.wait()
        cpb.wait()

        def add8(r, carry2):
            for rr in range(8):
                g_v[r * 8 + rr, :] = (ra_v[r * 8 + rr, 0:D]
                                      + rb_v[r * 8 + rr, D:2 * D])
            return carry2

        lax.fori_loop(0, GCH // 8, add8, 0)
        pltpu.sync_copy(g_v, g_hbm.at[pl.ds(base, GCH)])
        return carry

    lax.fori_loop(0, epw // GCH, chunk, 0)


def _gather_stage(T, i_idx, j_idx):
    E = i_idx.shape[0]
    k = pl.kernel(
        _gather_body,
        out_type=jax.ShapeDtypeStruct((E, D), jnp.float32),
        mesh=_SC_MESH,
        scratch_types=[
            pltpu.VMEM((GCH,), jnp.int32),
            pltpu.VMEM((GCH,), jnp.int32),
            pltpu.VMEM((GCH, 128), jnp.float32),
            pltpu.VMEM((GCH, 128), jnp.float32),
            pltpu.VMEM((GCH, D), jnp.float32),
            pltpu.SemaphoreType.DMA,
            pltpu.SemaphoreType.DMA,
        ],
    )
    return k(T, i_idx, j_idx)


def _scatter_body(inter_hbm, i_hbm, out_hbm,
                  idx_v, r16_v, r128_v, e_v, acc_sh, sem):
    cid = lax.axis_index("c")
    sid = lax.axis_index("s")
    E = inter_hbm.shape[0]
    npt = NPAD // N_TILES
    ept = E // N_SC // N_TILES

    def z128(r, c):
        e_v[r, :] = jnp.zeros((128,), jnp.float32)
        return c

    lax.fori_loop(0, 128, z128, 0)

    def zr(r, c):
        r128_v[r, :] = jnp.zeros((128,), jnp.float32)
        return c

    lax.fori_loop(0, SCH, zr, 0)

    def zcp(kk, c):
        pltpu.sync_copy(e_v, acc_sh.at[pl.ds(sid * npt + kk * 128, 128)])
        return c

    lax.fori_loop(0, npt // 128, zcp, 0)
    plsc.subcore_barrier()

    def chunk(c, carry):
        base = cid * (E // N_SC) + sid * ept + c * SCH
        pltpu.sync_copy(i_hbm.at[pl.ds(base, SCH)], idx_v)
        pltpu.sync_copy(inter_hbm.at[pl.ds(base, SCH)], r16_v)

        def rep8(r, c2):
            for rr in range(8):
                r128_v[r * 8 + rr, 0:D] = r16_v[r * 8 + rr, :]
            return c2

        lax.fori_loop(0, SCH // 8, rep8, 0)
        pltpu.sync_copy(r128_v, acc_sh.at[idx_v], add=True)
        return carry

    lax.fori_loop(0, ept // SCH, chunk, 0)
    plsc.subcore_barrier()

    def ecp(kk, c):
        pltpu.sync_copy(acc_sh.at[pl.ds(sid * npt + kk * 128, 128)], e_v)
        pltpu.sync_copy(
            e_v, out_hbm.at[pl.ds(cid * NPAD + sid * npt + kk * 128, 128)])
        return c

    lax.fori_loop(0, npt // 128, ecp, 0)


def _scatter_stage(inter, i_idx, n_atoms):
    k = pl.kernel(
        _scatter_body,
        out_type=jax.ShapeDtypeStruct((N_SC * NPAD, 128), jnp.float32),
        mesh=_SC_MESH,
        scratch_types=[
            pltpu.VMEM((SCH,), jnp.int32),
            pltpu.VMEM((SCH, D), jnp.float32),
            pltpu.VMEM((SCH, 128), jnp.float32),
            pltpu.VMEM((128, 128), jnp.float32),
            pltpu.VMEM_SHARED((NPAD, 128), jnp.float32),
            pltpu.SemaphoreType.DMA,
        ],
    )
    parts = k(inter, i_idx)
    return parts[:n_atoms], parts[NPAD:NPAD + n_atoms]


def _row(b):
    return b.reshape(1, -1)


def kernel(ind_1, elems, ind_2, diff, params):
    n_atoms = elems.shape[0]
    i_idx = jnp.asarray(ind_2[:, 0], dtype=jnp.int32)
    j_idx = jnp.asarray(ind_2[:, 1], dtype=jnp.int32)
    elems2d = elems.astype(jnp.int32).reshape(n_atoms, 1)

    fc = _compute_fc(diff)

    W = []
    for d in range(DEPTH):
        (wp0, bp0), (wp1, bp1) = params["pp"][d]
        (w1, b1), (w2, b2) = params["pi"][d]
        (wii0, _), (wii1, _) = params["ii"][d]
        (wo0, bo0), (wo1, bo1), (wof, bof) = params["out"][d]
        W.append(dict(
            wp0=wp0, bp0=_row(bp0), wp1=wp1, bp1=_row(bp1),
            w1a=w1[:D], w1b=w1[D:], b1=_row(b1),
            w2=w2, b2=_row(b2),
            wii0=wii0, wii1=wii1,
            wo0=wo0, bo0=_row(bo0), wo1=wo1, bo1=_row(bo1),
            wof=wof, bof=_row(bof),
        ))

    T = _atom0_stage(elems2d, W[0]["wp0"], W[0]["bp0"],
                     W[0]["wp1"], W[0]["bp1"],
                     W[0]["w1a"], W[0]["w1b"], W[0]["b1"])

    prop = jnp.zeros((n_atoms, D), jnp.float32)
    out_acc = jnp.zeros((n_atoms, 1), jnp.float32)
    for d in range(DEPTH):
        g = _gather_stage(T, i_idx, j_idx)
        inter = _edge_stage(g, fc, W[d]["w2"], W[d]["b2"],
                            W[d]["wii0"], W[d]["wii1"])
        np0, np1 = _scatter_stage(inter, i_idx, n_atoms)
        nxt = W[d + 1] if d + 1 < DEPTH else W[d]
        prop, out_acc, T = _atom_stage(
            d == 0, d == DEPTH - 1, np0, np1, prop, out_acc,
            W[d]["wo0"], W[d]["bo0"], W[d]["wo1"], W[d]["bo1"],
            W[d]["wof"], W[d]["bof"],
            nxt["wp0"], nxt["bp0"], nxt["wp1"], nxt["bp1"],
            nxt["w1a"], nxt["w1b"], nxt["b1"])

    return out_acc[:, 0]
